# Initial kernel scaffold; baseline (speedup 1.0000x reference)
import jax
import jax.numpy as jnp
from jax.experimental import pallas as pl


def kernel(embeds, edge_index, edge_weight, att_weight):
    raise NotImplementedError("write your pallas kernel here")


# speedup vs baseline: 4.5660x; 4.5660x over previous
import functools

import jax
import jax.numpy as jnp
from jax import lax
from jax.experimental import pallas as pl
from jax.experimental.pallas import tpu as pltpu
from jax.experimental.pallas import tpu_sc as plsc

_NC = 2
_NS = 16


def _sc_spmm(embeds, dst, src, w):
    N, D = embeds.shape
    E = dst.shape[0]
    NW = _NC * _NS
    EPT = E // NW
    C = 80
    K = EPT // C
    RPT = 624
    RB = 208
    T = RPT // RB
    TAIL = N - RPT * _NS

    mesh = plsc.VectorSubcoreMesh(core_axis_name="c", subcore_axis_name="s")

    @functools.partial(
        pl.kernel,
        mesh=mesh,
        out_type=jax.ShapeDtypeStruct((_NC, N, D), jnp.float32),
        scratch_types=[
            pltpu.VMEM((C,), jnp.int32),
            pltpu.VMEM((C,), jnp.int32),
            pltpu.VMEM((C,), jnp.float32),
            pltpu.VMEM((C, D), jnp.float32),
            pltpu.VMEM((RB, D), jnp.float32),
            pltpu.VMEM_SHARED((N, D), jnp.float32),
            pltpu.SemaphoreType.DMA,
        ],
    )
    def spmm(embeds_hbm, dst_hbm, src_hbm, w_hbm, out_hbm,
             src_v, dst_v, w_v, rows_v, buf_v, agg_sp, sem):
        cid = lax.axis_index("c")
        sid = lax.axis_index("s")
        wid = cid * _NS + sid
        row0 = sid * RPT

        def zb(i, carry):
            for j in range(D // 16):
                buf_v[i, pl.ds(j * 16, 16)] = jnp.zeros((16,), jnp.float32)
            return carry
        lax.fori_loop(0, RB, zb, 0)
        for t in range(T):
            pltpu.sync_copy(buf_v, agg_sp.at[pl.ds(row0 + t * RB, RB)])

        @pl.when(sid == _NS - 1)
        def _zero_tail():
            pltpu.sync_copy(buf_v.at[pl.ds(0, TAIL)],
                            agg_sp.at[pl.ds(RPT * _NS, TAIL)])
        plsc.subcore_barrier()

        ebase = wid * EPT

        def chunk(kk, carry):
            b = ebase + kk * C
            pltpu.sync_copy(src_hbm.at[pl.ds(b, C)], src_v)
            pltpu.sync_copy(dst_hbm.at[pl.ds(b, C)], dst_v)
            pltpu.sync_copy(w_hbm.at[pl.ds(b, C)], w_v)
            pltpu.async_copy(embeds_hbm.at[src_v], rows_v, sem).wait()

            def scale(g, c2):
                wv = w_v[pl.ds(g * 16, 16)]
                for l in range(16):
                    wi = wv[l]
                    i = g * 16 + l
                    for j in range(D // 16):
                        s = pl.ds(j * 16, 16)
                        rows_v[i, s] = rows_v[i, s] * wi
                return c2
            lax.fori_loop(0, C // 16, scale, 0)
            pltpu.sync_copy(rows_v, agg_sp.at[dst_v], add=True)
            return carry
        lax.fori_loop(0, K, chunk, 0)

        plsc.subcore_barrier()
        for t in range(T):
            pltpu.sync_copy(agg_sp.at[pl.ds(row0 + t * RB, RB)], buf_v)
            pltpu.sync_copy(buf_v, out_hbm.at[cid, pl.ds(row0 + t * RB, RB)])

        @pl.when(sid == _NS - 1)
        def _read_tail():
            pltpu.sync_copy(agg_sp.at[pl.ds(RPT * _NS, TAIL)],
                            buf_v.at[pl.ds(0, TAIL)])
            pltpu.sync_copy(buf_v.at[pl.ds(0, TAIL)],
                            out_hbm.at[cid, pl.ds(RPT * _NS, TAIL)])

    return spmm(embeds, dst, src, w)


def _tc_finish(partials, aw):
    N, D = partials.shape[1], partials.shape[2]

    def body(p_ref, a_ref, o_ref):
        agg = p_ref[0] + p_ref[1]
        aw_col = a_ref[...]
        scores = jnp.matmul(agg, aw_col)
        m = jnp.max(scores)
        e = jnp.exp(scores - m)
        att = e / jnp.sum(e)
        out = agg * att
        o_ref[...] = jnp.where(out >= 0, out, out * 0.2)

    return pl.pallas_call(
        body,
        out_shape=jax.ShapeDtypeStruct((N, D), jnp.float32),
    )(partials, aw)


def kernel(embeds, edge_index, edge_weight, att_weight):
    dst = edge_index[0]
    src = edge_index[1]
    partials = _sc_spmm(embeds, dst, src, edge_weight)
    return _tc_finish(partials, att_weight)
